# baseline (speedup 1.0000x reference)
import functools

import jax
import jax.numpy as jnp
from jax import lax
from jax.experimental import pallas as pl
from jax.experimental.pallas import tpu as pltpu
from jax.experimental.pallas import tpu_sc as plsc

_LANES = 16
_NSLOT = 5


def _embed_kernel(n_b, n_s, d, seq_vocab):
    info = plsc.get_sparse_core_info()
    nc, ns = info.num_cores, info.num_subcores
    nw = nc * ns
    n_tokens = n_b * n_s
    chunk = 128
    per_w = n_tokens // nw
    n_chunks = per_w // chunk
    assert per_w % chunk == 0 and n_tokens % nw == 0
    assert n_chunks % _NSLOT == 0 and n_chunks >= 2 * _NSLOT
    assert n_b % chunk == 0

    mesh = plsc.VectorSubcoreMesh(core_axis_name="c", subcore_axis_name="s")

    @functools.partial(
        pl.kernel,
        mesh=mesh,
        out_type=jax.ShapeDtypeStruct((n_s, d, n_b), jnp.float32),
        scratch_types=[
            pltpu.VMEM((_NSLOT, chunk), jnp.int32),
            pltpu.VMEM((_NSLOT, chunk), jnp.int32),
            pltpu.VMEM((_NSLOT, chunk), jnp.int32),
            pltpu.VMEM((_NSLOT, chunk), jnp.int32),
            pltpu.VMEM((_NSLOT, chunk, d), jnp.bfloat16),
            pltpu.VMEM((_NSLOT, d, chunk + 8), jnp.float32),
            pltpu.SemaphoreType.DMA((_NSLOT,)),
            pltpu.SemaphoreType.DMA((_NSLOT,)),
            pltpu.SemaphoreType.DMA((_NSLOT,)),
            pltpu.SemaphoreType.DMA((_NSLOT,)),
        ],
        compiler_params=pltpu.CompilerParams(use_tc_tiling_on_sc=False,
                                             needs_layout_passes=False),
    )
    def k(ids_hbm, seg_hbm, pos_hbm, tok_tab, comb_tab, out_hbm,
          idxs, segs, poss, cidxs, rows, obufs,
          sem_idx, sem_tok, sem_comb, sem_out):
        wid = lax.axis_index("s") * nc + lax.axis_index("c")
        base = wid * per_w

        def load_idx(g, b):
            off = base + g * chunk
            pltpu.async_copy(ids_hbm.at[pl.ds(off, chunk)], idxs.at[b],
                             sem_idx.at[b])
            pltpu.async_copy(seg_hbm.at[pl.ds(off, chunk)], segs.at[b],
                             sem_idx.at[b])
            pltpu.async_copy(pos_hbm.at[pl.ds(off, chunk)], poss.at[b],
                             sem_idx.at[b])

        def wait_idx(b):
            for _ in range(3):
                pltpu.make_async_copy(ids_hbm.at[pl.ds(base, chunk)],
                                      idxs.at[b], sem_idx.at[b]).wait()

        def compute_cidx(b):
            for i in range(chunk // _LANES):
                sl = pl.ds(i * _LANES, _LANES)
                cidxs[b, sl] = segs[b, sl] * seq_vocab + poss[b, sl]

        def fire_tok(b):
            pltpu.async_copy(tok_tab.at[idxs.at[b]], rows.at[b],
                             sem_tok.at[b])

        def wait_tok(b):
            pltpu.make_async_copy(tok_tab.at[idxs.at[b]], rows.at[b],
                                  sem_tok.at[b]).wait()

        def fire_comb(b):
            pltpu.async_copy(comb_tab.at[cidxs.at[b]], rows.at[b],
                             sem_comb.at[b], add=True)

        def wait_comb(b):
            pltpu.make_async_copy(comb_tab.at[cidxs.at[b]], rows.at[b],
                                  sem_comb.at[b]).wait()

        def fire_out(g, b):
            flat = base + g * chunk
            s = flat // n_b
            b0 = flat % n_b
            pltpu.async_copy(obufs.at[b, :, pl.ds(0, chunk)],
                             out_hbm.at[s, :, pl.ds(b0, chunk)],
                             sem_out.at[b])

        def wait_out(b):
            pltpu.make_async_copy(obufs.at[b, :, pl.ds(0, chunk)],
                                  out_hbm.at[0, :, pl.ds(0, chunk)],
                                  sem_out.at[b]).wait()

        def transpose_pass(b):
            obuf_b = obufs.at[b]

            @plsc.parallel_loop(0, chunk // _LANES, 1, unroll=4)
            def _(jb):
                eva = [lax.iota(jnp.int32, _LANES) * 2 + eg2 * 2 * _LANES
                       for eg2 in range(d // (2 * _LANES))]
                evb = [v + 1 for v in eva]
                for jj in range(_LANES):
                    j = jb * _LANES + jj
                    jfull = jnp.full((_LANES,), 0, jnp.int32) + j
                    for eg2 in range(d // (2 * _LANES)):
                        v32 = rows[b, j, pl.ds(eg2 * 2 * _LANES, 2 * _LANES)]
                        va, vb = plsc.unpack(v32,
                                             format=plsc.PackFormat.INTERLEAVED)
                        plsc.store_scatter(obuf_b, [eva[eg2], jfull], va)
                        plsc.store_scatter(obuf_b, [evb[eg2], jfull], vb)

        load_idx(0, 0)
        load_idx(1, 1)
        load_idx(2, 2)
        wait_idx(0)
        compute_cidx(0)
        fire_tok(0)
        wait_idx(1)
        compute_cidx(1)
        fire_tok(1)
        wait_tok(0)
        fire_comb(0)

        def body(gq, carry):
            for b in range(_NSLOT):
                g = gq * _NSLOT + b
                c1 = g + 2

                @pl.when(c1 < n_chunks)
                def _():
                    b1 = (b + 2) % _NSLOT
                    wait_idx(b1)
                    compute_cidx(b1)
                    fire_tok(b1)

                @pl.when(g + 1 < n_chunks)
                def _():
                    b2 = (b + 1) % _NSLOT
                    wait_tok(b2)
                    fire_comb(b2)

                wait_comb(b)

                @pl.when(g >= _NSLOT)
                def _():
                    wait_out(b)

                transpose_pass(b)
                fire_out(g, b)

                @pl.when(g + 3 < n_chunks)
                def _():
                    load_idx(g + 3, (b + 3) % _NSLOT)
            return carry

        lax.fori_loop(0, n_chunks // _NSLOT, body, 0)
        for b in range(_NSLOT):
            wait_out(b)

    return k


def kernel(input_ids, segment_ids, position_ids, token_table,
           segment_table, position_table):
    b, s = input_ids.shape
    d = token_table.shape[1]
    seq_vocab = position_table.shape[0]

    if position_ids is None:
        position_ids = jnp.broadcast_to(
            jnp.arange(s, dtype=input_ids.dtype)[None, :], (b, s))

    ids = input_ids.T.reshape(-1).astype(jnp.int32)
    seg = segment_ids.T.reshape(-1).astype(jnp.int32)
    pos = position_ids.T.reshape(-1).astype(jnp.int32)
    comb = (segment_table[:, None, :] + position_table[None, :, :]).reshape(
        segment_table.shape[0] * seq_vocab, d)

    tok_bf = token_table.astype(jnp.bfloat16)
    comb_bf = comb.astype(jnp.bfloat16)
    out_t = _embed_kernel(b, s, d, seq_vocab)(ids, seg, pos, tok_bf, comb_bf)
    return out_t.transpose(2, 0, 1)


# speedup vs baseline: 1.1016x; 1.1016x over previous
import functools

import jax
import jax.numpy as jnp
from jax import lax
from jax.experimental import pallas as pl
from jax.experimental.pallas import tpu as pltpu
from jax.experimental.pallas import tpu_sc as plsc

_LANES = 16
_NSLOT = 5


def _embed_kernel(n_b, n_s, d, seq_vocab):
    info = plsc.get_sparse_core_info()
    nc, ns = info.num_cores, info.num_subcores
    nw = nc * ns
    n_tokens = n_b * n_s
    chunk = 128
    per_w = n_tokens // nw
    n_chunks = per_w // chunk
    assert per_w % chunk == 0 and n_tokens % nw == 0
    assert n_chunks % _NSLOT == 0 and n_chunks >= 2 * _NSLOT
    assert n_b % chunk == 0

    mesh = plsc.VectorSubcoreMesh(core_axis_name="c", subcore_axis_name="s")

    @functools.partial(
        pl.kernel,
        mesh=mesh,
        out_type=jax.ShapeDtypeStruct((n_s, d, n_b), jnp.float32),
        scratch_types=[
            pltpu.VMEM((_NSLOT, chunk), jnp.int32),
            pltpu.VMEM((_NSLOT, chunk), jnp.int32),
            pltpu.VMEM((_NSLOT, chunk), jnp.int32),
            pltpu.VMEM((_NSLOT, chunk), jnp.int32),
            pltpu.VMEM((_NSLOT, chunk, d), jnp.float32),
            pltpu.VMEM((_NSLOT, d, chunk + 8), jnp.float32),
            pltpu.SemaphoreType.DMA((_NSLOT,)),
            pltpu.SemaphoreType.DMA((_NSLOT,)),
            pltpu.SemaphoreType.DMA((_NSLOT,)),
            pltpu.SemaphoreType.DMA((_NSLOT,)),
        ],
        compiler_params=pltpu.CompilerParams(use_tc_tiling_on_sc=False,
                                             needs_layout_passes=False),
    )
    def k(ids_hbm, seg_hbm, pos_hbm, tok_tab, comb_tab, out_hbm,
          idxs, segs, poss, cidxs, rows, obufs,
          sem_idx, sem_tok, sem_comb, sem_out):
        wid = lax.axis_index("s") * nc + lax.axis_index("c")
        base = wid * per_w

        def load_idx(g, b):
            off = base + g * chunk
            pltpu.async_copy(ids_hbm.at[pl.ds(off, chunk)], idxs.at[b],
                             sem_idx.at[b])
            pltpu.async_copy(seg_hbm.at[pl.ds(off, chunk)], segs.at[b],
                             sem_idx.at[b])
            pltpu.async_copy(pos_hbm.at[pl.ds(off, chunk)], poss.at[b],
                             sem_idx.at[b])

        def wait_idx(b):
            for _ in range(3):
                pltpu.make_async_copy(ids_hbm.at[pl.ds(base, chunk)],
                                      idxs.at[b], sem_idx.at[b]).wait()

        def compute_cidx(b):
            for i in range(chunk // _LANES):
                sl = pl.ds(i * _LANES, _LANES)
                cidxs[b, sl] = segs[b, sl] * seq_vocab + poss[b, sl]

        def fire_tok(b):
            pltpu.async_copy(tok_tab.at[idxs.at[b]], rows.at[b],
                             sem_tok.at[b])

        def wait_tok(b):
            pltpu.make_async_copy(tok_tab.at[idxs.at[b]], rows.at[b],
                                  sem_tok.at[b]).wait()

        def fire_comb(b):
            pltpu.async_copy(comb_tab.at[cidxs.at[b]], rows.at[b],
                             sem_comb.at[b], add=True)

        def wait_comb(b):
            pltpu.make_async_copy(comb_tab.at[cidxs.at[b]], rows.at[b],
                                  sem_comb.at[b]).wait()

        def fire_out(g, b):
            flat = base + g * chunk
            s = flat // n_b
            b0 = flat % n_b
            pltpu.async_copy(obufs.at[b, :, pl.ds(0, chunk)],
                             out_hbm.at[s, :, pl.ds(b0, chunk)],
                             sem_out.at[b])

        def wait_out(b):
            pltpu.make_async_copy(obufs.at[b, :, pl.ds(0, chunk)],
                                  out_hbm.at[0, :, pl.ds(0, chunk)],
                                  sem_out.at[b]).wait()

        def transpose_pass(b):
            obuf_b = obufs.at[b]

            @plsc.parallel_loop(0, chunk // _LANES, 1, unroll=4)
            def _(jb):
                evs = [lax.iota(jnp.int32, _LANES) + eg * _LANES
                       for eg in range(d // _LANES)]
                for jj in range(_LANES):
                    j = jb * _LANES + jj
                    jfull = jnp.full((_LANES,), 0, jnp.int32) + j
                    for eg in range(d // _LANES):
                        v = rows[b, j, pl.ds(eg * _LANES, _LANES)]
                        plsc.store_scatter(obuf_b, [evs[eg], jfull], v)

        load_idx(0, 0)
        load_idx(1, 1)
        load_idx(2, 2)
        wait_idx(0)
        compute_cidx(0)
        fire_tok(0)
        wait_idx(1)
        compute_cidx(1)
        fire_tok(1)
        wait_tok(0)
        fire_comb(0)

        def body(gq, carry):
            for b in range(_NSLOT):
                g = gq * _NSLOT + b
                c1 = g + 2

                @pl.when(c1 < n_chunks)
                def _():
                    b1 = (b + 2) % _NSLOT
                    wait_idx(b1)
                    compute_cidx(b1)
                    fire_tok(b1)

                @pl.when(g + 1 < n_chunks)
                def _():
                    b2 = (b + 1) % _NSLOT
                    wait_tok(b2)
                    fire_comb(b2)

                wait_comb(b)

                @pl.when(g >= _NSLOT)
                def _():
                    wait_out(b)

                transpose_pass(b)
                fire_out(g, b)

                @pl.when(g + 3 < n_chunks)
                def _():
                    load_idx(g + 3, (b + 3) % _NSLOT)
            return carry

        lax.fori_loop(0, n_chunks // _NSLOT, body, 0)
        for b in range(_NSLOT):
            wait_out(b)

    return k


def kernel(input_ids, segment_ids, position_ids, token_table,
           segment_table, position_table):
    b, s = input_ids.shape
    d = token_table.shape[1]
    seq_vocab = position_table.shape[0]

    if position_ids is None:
        position_ids = jnp.broadcast_to(
            jnp.arange(s, dtype=input_ids.dtype)[None, :], (b, s))

    ids = input_ids.T.reshape(-1).astype(jnp.int32)
    seg = segment_ids.T.reshape(-1).astype(jnp.int32)
    pos = position_ids.T.reshape(-1).astype(jnp.int32)
    comb = (segment_table[:, None, :] + position_table[None, :, :]).reshape(
        segment_table.shape[0] * seq_vocab, d)

    out_t = _embed_kernel(b, s, d, seq_vocab)(ids, seg, pos, token_table, comb)
    return out_t.transpose(2, 0, 1)
